# baseline (speedup 1.0000x reference)
import functools

import jax
import jax.numpy as jnp
from jax import lax
from jax.experimental import pallas as pl
from jax.experimental.pallas import tpu as pltpu
from jax.experimental.pallas import tpu_sc as plsc

_ROWS = 2


def _make_sc_kernel(B, H, D, n_workers, num_cores):
    R = _ROWS
    b_per_w = B // n_workers
    n = b_per_w // R
    mesh = plsc.VectorSubcoreMesh(core_axis_name="c", subcore_axis_name="s")

    @functools.partial(
        pl.kernel,
        mesh=mesh,
        compiler_params=pltpu.CompilerParams(use_tc_tiling_on_sc=False),
        out_type=jax.ShapeDtypeStruct((B, H, D), jnp.float32),
        scratch_types=[
            pltpu.VMEM((R, H), jnp.int32),
            pltpu.VMEM((R, H), jnp.int32),
            pltpu.VMEM((R, H, D), jnp.float32),
            pltpu.VMEM((R, H, D), jnp.float32),
            pltpu.VMEM((R, H, D), jnp.float32),
            pltpu.VMEM((R, H, D), jnp.float32),
            pltpu.VMEM((R, H, D), jnp.float32),
            pltpu.VMEM((R, H, D), jnp.float32),
            pltpu.VMEM((R, H, D), jnp.float32),
            pltpu.VMEM((R, H, D), jnp.float32),
            pltpu.SemaphoreType.DMA,
            pltpu.SemaphoreType.DMA,
            pltpu.SemaphoreType.DMA,
            pltpu.SemaphoreType.DMA,
            pltpu.SemaphoreType.DMA,
        ],
    )
    def sc_kernel(idx_hbm, w_hbm, s_hbm, eps_hbm, out_hbm,
                  idx0, idx1, mu0, mu1, lv0, lv1, eps0, eps1, o0, o1,
                  sem_idx, sem_mu, sem_lv, sem_eps, sem_out):
        wid = lax.axis_index("s") * num_cores + lax.axis_index("c")
        base_w = wid * b_per_w

        idx_b = (idx0, idx1)
        mu_b = (mu0, mu1)
        lv_b = (lv0, lv1)
        eps_b = (eps0, eps1)
        o_b = (o0, o1)

        def idx_start(ci, s):
            src = idx_hbm.at[pl.ds(base_w + ci * R, R)]
            pltpu.make_async_copy(src, idx_b[s], sem_idx).start()

        def idx_wait(s):
            src = idx_hbm.at[pl.ds(0, R)]
            pltpu.make_async_copy(src, idx_b[s], sem_idx).wait()

        def gather_start(ci, s):
            for r2 in range(R):
                row_idx = idx_b[s].at[r2]
                pltpu.make_async_copy(
                    w_hbm.at[row_idx], mu_b[s].at[r2], sem_mu).start()
                pltpu.make_async_copy(
                    s_hbm.at[row_idx], lv_b[s].at[r2], sem_lv).start()
            src = eps_hbm.at[pl.ds(base_w + ci * R, R)]
            pltpu.make_async_copy(src, eps_b[s], sem_eps).start()

        def gather_wait(s):
            for r2 in range(R):
                row_idx = idx_b[s].at[r2]
                pltpu.make_async_copy(
                    w_hbm.at[row_idx], mu_b[s].at[r2], sem_mu).wait()
                pltpu.make_async_copy(
                    s_hbm.at[row_idx], lv_b[s].at[r2], sem_lv).wait()
            src = eps_hbm.at[pl.ds(0, R)]
            pltpu.make_async_copy(src, eps_b[s], sem_eps).wait()

        def out_start(ci, s):
            dst = out_hbm.at[pl.ds(base_w + ci * R, R)]
            pltpu.make_async_copy(o_b[s], dst, sem_out).start()

        def out_wait(s):
            dst = out_hbm.at[pl.ds(0, R)]
            pltpu.make_async_copy(o_b[s], dst, sem_out).wait()

        def compute(s):
            mu_v, lv_v, eps_v, o_v = mu_b[s], lv_b[s], eps_b[s], o_b[s]

            @plsc.parallel_loop(0, H, 1, unroll=4)
            def row_body(r):
                for r2 in range(R):
                    for j in range(D // 16):
                        sl = pl.ds(j * 16, 16)
                        std = jnp.exp(0.5 * lv_v[r2, r, sl])
                        o_v[r2, r, sl] = eps_v[r2, r, sl] * std + mu_v[r2, r, sl]

        idx_start(0, 0)
        idx_wait(0)
        gather_start(0, 0)
        idx_start(1, 1)

        idx_wait(1)
        gather_start(1, 1)
        gather_wait(0)
        idx_start(2, 0)
        compute(0)
        out_start(0, 0)

        idx_wait(0)
        gather_start(2, 0)
        gather_wait(1)
        idx_start(3, 1)
        compute(1)
        out_start(1, 1)

        def pair_body(p, carry):
            for k in range(2):
                ci = 2 + 2 * p + k
                idx_wait(1 - k)
                gather_start(ci + 1, 1 - k)
                gather_wait(k)
                idx_start(ci + 2, k)
                out_wait(k)
                compute(k)
                out_start(ci, k)
            return carry

        lax.fori_loop(0, (n - 4) // 2, pair_body, 0)

        idx_wait(1)
        gather_start(n - 1, 1)
        gather_wait(0)
        out_wait(0)
        compute(0)
        out_start(n - 2, 0)

        gather_wait(1)
        out_wait(1)
        compute(1)
        out_start(n - 1, 1)

        out_wait(0)
        out_wait(1)

    return sc_kernel


def kernel(indices, weight, spread, eps):
    B, H = indices.shape
    V, D = weight.shape

    info = plsc.get_sparse_core_info()
    n_workers = info.num_cores * info.num_subcores

    sc_kernel = _make_sc_kernel(B, H, D, n_workers, info.num_cores)
    return sc_kernel(indices, weight, spread, eps)


# speedup vs baseline: 1.2635x; 1.2635x over previous
import functools

import jax
import jax.numpy as jnp
from jax import lax
from jax.experimental import pallas as pl
from jax.experimental.pallas import tpu as pltpu
from jax.experimental.pallas import tpu_sc as plsc

_ROWS = 2


def _make_sc_kernel(B, H, D, n_workers, num_cores):
    R = _ROWS
    HP = (H + 7) // 8 * 8
    b_per_w = B // n_workers
    n = b_per_w // R
    mesh = plsc.VectorSubcoreMesh(core_axis_name="c", subcore_axis_name="s")

    @functools.partial(
        pl.kernel,
        mesh=mesh,
        compiler_params=pltpu.CompilerParams(use_tc_tiling_on_sc=True),
        out_type=jax.ShapeDtypeStruct((B, H, D), jnp.float32),
        scratch_types=[
            pltpu.VMEM((R, H), jnp.int32),
            pltpu.VMEM((R, H), jnp.int32),
            pltpu.VMEM((R, HP, 2 * D), jnp.float32),
            pltpu.VMEM((R, HP, 2 * D), jnp.float32),
            pltpu.VMEM((R, H, D), jnp.float32),
            pltpu.VMEM((R, H, D), jnp.float32),
            pltpu.VMEM((R, H, D), jnp.float32),
            pltpu.VMEM((R, H, D), jnp.float32),
            pltpu.SemaphoreType.DMA,
            pltpu.SemaphoreType.DMA,
            pltpu.SemaphoreType.DMA,
            pltpu.SemaphoreType.DMA,
        ],
    )
    def sc_kernel(idx_hbm, pk_hbm, eps_hbm, out_hbm,
                  idx0, idx1, pk0, pk1, eps0, eps1, o0, o1,
                  sem_idx, sem_pk, sem_eps, sem_out):
        wid = lax.axis_index("s") * num_cores + lax.axis_index("c")
        base_w = wid * b_per_w

        idx_b = (idx0, idx1)
        pk_b = (pk0, pk1)
        eps_b = (eps0, eps1)
        o_b = (o0, o1)

        def idx_start(ci, s):
            src = idx_hbm.at[pl.ds(base_w + ci * R, R)]
            pltpu.make_async_copy(src, idx_b[s], sem_idx).start()

        def idx_wait(s):
            src = idx_hbm.at[pl.ds(0, R)]
            pltpu.make_async_copy(src, idx_b[s], sem_idx).wait()

        def gather_start(ci, s):
            for r2 in range(R):
                row_idx = idx_b[s].at[r2]
                dst = pk_b[s].at[r2, pl.ds(0, H)]
                pltpu.make_async_copy(pk_hbm.at[row_idx], dst, sem_pk).start()
            src = eps_hbm.at[pl.ds(base_w + ci * R, R)]
            pltpu.make_async_copy(src, eps_b[s], sem_eps).start()

        def gather_wait(s):
            for r2 in range(R):
                row_idx = idx_b[s].at[r2]
                dst = pk_b[s].at[r2, pl.ds(0, H)]
                pltpu.make_async_copy(pk_hbm.at[row_idx], dst, sem_pk).wait()
            src = eps_hbm.at[pl.ds(0, R)]
            pltpu.make_async_copy(src, eps_b[s], sem_eps).wait()

        def out_start(ci, s):
            dst = out_hbm.at[pl.ds(base_w + ci * R, R)]
            pltpu.make_async_copy(o_b[s], dst, sem_out).start()

        def out_wait(s):
            dst = out_hbm.at[pl.ds(0, R)]
            pltpu.make_async_copy(o_b[s], dst, sem_out).wait()

        def compute(s):
            pk_v, eps_v, o_v = pk_b[s], eps_b[s], o_b[s]

            @plsc.parallel_loop(0, H, 1, unroll=4)
            def row_body(r):
                for r2 in range(R):
                    for j in range(D // 16):
                        mu = pk_v[r2, r, pl.ds(j * 16, 16)]
                        lv = pk_v[r2, r, pl.ds(D + j * 16, 16)]
                        sl = pl.ds(j * 16, 16)
                        std = jnp.exp(0.5 * lv)
                        o_v[r2, r, sl] = eps_v[r2, r, sl] * std + mu

        idx_start(0, 0)
        idx_wait(0)
        gather_start(0, 0)
        idx_start(1, 1)

        idx_wait(1)
        gather_start(1, 1)
        gather_wait(0)
        idx_start(2, 0)
        compute(0)
        out_start(0, 0)

        idx_wait(0)
        gather_start(2, 0)
        gather_wait(1)
        idx_start(3, 1)
        compute(1)
        out_start(1, 1)

        def pair_body(p, carry):
            for k in range(2):
                ci = 2 + 2 * p + k
                idx_wait(1 - k)
                gather_start(ci + 1, 1 - k)
                gather_wait(k)
                idx_start(ci + 2, k)
                out_wait(k)
                compute(k)
                out_start(ci, k)
            return carry

        lax.fori_loop(0, (n - 4) // 2, pair_body, 0)

        idx_wait(1)
        gather_start(n - 1, 1)
        gather_wait(0)
        out_wait(0)
        compute(0)
        out_start(n - 2, 0)

        gather_wait(1)
        out_wait(1)
        compute(1)
        out_start(n - 1, 1)

        out_wait(0)
        out_wait(1)

    return sc_kernel


def kernel(indices, weight, spread, eps):
    B, H = indices.shape
    V, D = weight.shape

    packed = jnp.concatenate([weight, spread], axis=1)

    info = plsc.get_sparse_core_info()
    n_workers = info.num_cores * info.num_subcores

    sc_kernel = _make_sc_kernel(B, H, D, n_workers, info.num_cores)
    return sc_kernel(indices, packed, eps)
